# baseline (speedup 1.0000x reference)
import functools

import jax
import jax.numpy as jnp
from jax import lax
from jax.experimental import pallas as pl
from jax.experimental.pallas import tpu as pltpu
from jax.experimental.pallas import tpu_sc as plsc

N = 10000
F = 128
E = 320000

NC = 2
NS = 16
NW = NC * NS
K = 128
NCH = 80
NCHH = NCH // 2
PAD_E = NW * NCH * K
NPAD = 10240
DR = NPAD // 128
ROWS_PT = NPAD // NS

_mesh = plsc.VectorSubcoreMesh(core_axis_name="c", subcore_axis_name="s")


@functools.partial(
    pl.kernel,
    out_type=jax.ShapeDtypeStruct((NC, DR, 128), jnp.float32),
    mesh=_mesh,
    compiler_params=pltpu.CompilerParams(needs_layout_passes=False),
    scratch_types=[
        pltpu.VMEM((NCH, K), jnp.int32),
        pltpu.VMEM((DR, 128), jnp.float32),
        pltpu.VMEM((DR,), jnp.int32),
        pltpu.VMEM_SHARED((DR, 128), jnp.float32),
    ],
)
def _sc_degree(col_hbm, z_hbm, deg_out, colv, hist, idxv, deg_sh):
    cid = lax.axis_index("c")
    sid = lax.axis_index("s")
    wid = sid * NC + cid

    pltpu.sync_copy(col_hbm.at[wid], colv)
    zeros16 = jnp.zeros((16,), jnp.float32)

    def zbody(r, carry):
        for j in range(8):
            hist[r, pl.ds(j * 16, 16)] = zeros16
        return carry

    lax.fori_loop(0, DR, zbody, 0)

    @pl.when(sid == 0)
    def _():
        pltpu.sync_copy(z_hbm, deg_sh)
    for j in range(DR // 16):
        idxv[pl.ds(j * 16, 16)] = lax.iota(jnp.int32, 16) + jnp.int32(j * 16)

    ones16 = jnp.ones((16,), jnp.float32)

    def body(j, carry):
        for l in range(8):
            idx = colv[j, pl.ds(l * 16, 16)]
            plsc.addupdate_scatter(hist, [idx // 128, idx % 128], ones16)
        return carry

    lax.fori_loop(0, NCH, body, 0)
    plsc.subcore_barrier()
    pltpu.sync_copy(hist, deg_sh.at[idxv], add=True)
    plsc.subcore_barrier()

    @pl.when(sid == 0)
    def _():
        pltpu.sync_copy(deg_sh, deg_out.at[cid])


@functools.partial(
    pl.kernel,
    out_type=jax.ShapeDtypeStruct((NC, NPAD, F), jnp.float32),
    mesh=_mesh,
    scratch_types=[
        pltpu.VMEM((NCHH, K), jnp.int32),
        pltpu.VMEM((NCHH, K), jnp.int32),
        pltpu.VMEM((K, F), jnp.float32),
        pltpu.VMEM((K, F), jnp.float32),
        pltpu.VMEM_SHARED((NPAD, F), jnp.float32),
        pltpu.SemaphoreType.DMA,
        pltpu.SemaphoreType.DMA,
    ],
)
def _sc_edge_pass(gs_hbm, row_hbm, col_hbm, z_hbm, acc_out,
                  rowv, colv, rbuf0, rbuf1, acc_sh, sem0, sem1):
    cid = lax.axis_index("c")
    sid = lax.axis_index("s")
    wid = sid * NC + cid
    rbufs = (rbuf0, rbuf1)
    sems = (sem0, sem1)

    @pl.when(sid == 0)
    def _():
        pltpu.sync_copy(z_hbm, acc_sh)

    plsc.subcore_barrier()

    for p in range(2):
        pltpu.sync_copy(row_hbm.at[wid, pl.ds(p * NCHH, NCHH)], rowv)
        pltpu.sync_copy(col_hbm.at[wid, pl.ds(p * NCHH, NCHH)], colv)
        pltpu.async_copy(gs_hbm.at[rowv.at[0]], rbuf0, sem0)
        pltpu.async_copy(gs_hbm.at[rowv.at[1]], rbuf1, sem1)

        def body(j0, carry):
            for b in range(2):
                j = 2 * j0 + b
                pltpu.make_async_copy(
                    gs_hbm.at[rowv.at[j]], rbufs[b], sems[b]).wait()
                pltpu.sync_copy(rbufs[b], acc_sh.at[colv.at[j]], add=True)

                @pl.when(j + 2 < NCHH)
                def _():
                    pltpu.async_copy(
                        gs_hbm.at[rowv.at[j + 2]], rbufs[b], sems[b])
            return carry

        lax.fori_loop(0, NCHH // 2, body, 0)

    plsc.subcore_barrier()
    pltpu.sync_copy(
        acc_sh.at[pl.ds(sid * ROWS_PT, ROWS_PT)],
        acc_out.at[cid, pl.ds(sid * ROWS_PT, ROWS_PT)],
    )


BN = 1000


def _tc_scale_matmul(x, d0, d1, Wc, Wl):
    def body(x_ref, d0_ref, d1_ref, wc_ref, wl_ref, gs_ref, dinv_ref):
        deg = d0_ref[...] + d1_ref[...] + 1.0
        dinv = lax.rsqrt(deg)
        g = jnp.dot(x_ref[...], wc_ref[...], preferred_element_type=jnp.float32)
        g = jnp.dot(g, wl_ref[...], preferred_element_type=jnp.float32)
        gs_ref[...] = g * dinv
        dinv_ref[...] = dinv

    return pl.pallas_call(
        body,
        grid=(N // BN,),
        in_specs=[
            pl.BlockSpec((BN, F), lambda i: (i, 0)),
            pl.BlockSpec((BN, 1), lambda i: (i, 0)),
            pl.BlockSpec((BN, 1), lambda i: (i, 0)),
            pl.BlockSpec((F, F), lambda i: (0, 0)),
            pl.BlockSpec((F, F), lambda i: (0, 0)),
        ],
        out_specs=[
            pl.BlockSpec((BN, F), lambda i: (i, 0)),
            pl.BlockSpec((BN, 1), lambda i: (i, 0)),
        ],
        out_shape=[
            jax.ShapeDtypeStruct((N, F), jnp.float32),
            jax.ShapeDtypeStruct((N, 1), jnp.float32),
        ],
    )(x, d0, d1, Wc, Wl)


def _tc_combine(a0, a1, gs, dinv, Wl, bc2, bl2):
    def body(a0_ref, a1_ref, gs_ref, dinv_ref, wl_ref, bc_ref, bl_ref, o_ref):
        acc = a0_ref[...] + a1_ref[...] + gs_ref[...]
        bf = jnp.dot(bc_ref[...], wl_ref[...],
                     preferred_element_type=jnp.float32) + bl_ref[...]
        o_ref[...] = acc * dinv_ref[...] + bf

    return pl.pallas_call(
        body,
        grid=(N // BN,),
        in_specs=[
            pl.BlockSpec((BN, F), lambda i: (i, 0)),
            pl.BlockSpec((BN, F), lambda i: (i, 0)),
            pl.BlockSpec((BN, F), lambda i: (i, 0)),
            pl.BlockSpec((BN, 1), lambda i: (i, 0)),
            pl.BlockSpec((F, F), lambda i: (0, 0)),
            pl.BlockSpec((1, F), lambda i: (0, 0)),
            pl.BlockSpec((1, F), lambda i: (0, 0)),
        ],
        out_specs=pl.BlockSpec((BN, F), lambda i: (i, 0)),
        out_shape=jax.ShapeDtypeStruct((N, F), jnp.float32),
    )(a0, a1, gs, dinv, Wl, bc2, bl2)


@jax.jit
def kernel(x, edge_index, W_conv, b_conv, W_lin, b_lin):
    row = edge_index[0]
    col = edge_index[1]
    pad = PAD_E - E
    rowp = jnp.concatenate(
        [row, jnp.zeros((pad,), jnp.int32)]).reshape(NW, NCH, K)
    colp = jnp.concatenate(
        [col, jnp.full((pad,), N, jnp.int32)]).reshape(NW, NCH, K)

    zdeg = jnp.zeros((DR, 128), jnp.float32)
    zrows = jnp.zeros((NPAD, F), jnp.float32)

    deg2 = _sc_degree(colp, zdeg)
    gs, dinv = _tc_scale_matmul(x, deg2[0].reshape(NPAD, 1),
                                deg2[1].reshape(NPAD, 1), W_conv, W_lin)
    acc2 = _sc_edge_pass(gs, rowp, colp, zrows)
    out = _tc_combine(acc2[0], acc2[1], gs, dinv, W_lin,
                      b_conv.reshape(1, F), b_lin.reshape(1, F))
    return out


# speedup vs baseline: 1.3292x; 1.3292x over previous
import functools

import jax
import jax.numpy as jnp
from jax import lax
from jax.experimental import pallas as pl
from jax.experimental.pallas import tpu as pltpu
from jax.experimental.pallas import tpu_sc as plsc

N = 10000
F = 128
E = 320000

NC = 2
NS = 16
NW = NC * NS
K = 128
NCH = -(-E // (NW * K))
PAD_E = NW * NCH * K
NPAD = 10240
DR = NPAD // 128
ROWS_PT = NPAD // NS

_mesh = plsc.VectorSubcoreMesh(core_axis_name="c", subcore_axis_name="s")


@functools.partial(
    pl.kernel,
    out_type=jax.ShapeDtypeStruct((NC, DR, 128), jnp.float32),
    mesh=_mesh,
    compiler_params=pltpu.CompilerParams(needs_layout_passes=False),
    scratch_types=[
        pltpu.VMEM((NCH, K), jnp.int32),
        pltpu.VMEM((DR, 128), jnp.float32),
        pltpu.VMEM((DR,), jnp.int32),
        pltpu.VMEM_SHARED((DR, 128), jnp.float32),
    ],
)
def _sc_degree(col_hbm, z_hbm, deg_out, colv, hist, idxv, deg_sh):
    cid = lax.axis_index("c")
    sid = lax.axis_index("s")
    wid = sid * NC + cid

    pltpu.sync_copy(col_hbm.at[wid], colv)
    zeros16 = jnp.zeros((16,), jnp.float32)

    def zbody(r, carry):
        for j in range(8):
            hist[r, pl.ds(j * 16, 16)] = zeros16
        return carry

    lax.fori_loop(0, DR, zbody, 0)

    @pl.when(sid == 0)
    def _():
        pltpu.sync_copy(z_hbm, deg_sh)
    for j in range(DR // 16):
        idxv[pl.ds(j * 16, 16)] = lax.iota(jnp.int32, 16) + jnp.int32(j * 16)

    ones16 = jnp.ones((16,), jnp.float32)

    def body(j, carry):
        for l in range(8):
            idx = colv[j, pl.ds(l * 16, 16)]
            plsc.addupdate_scatter(hist, [idx // 128, idx % 128], ones16)
        return carry

    lax.fori_loop(0, NCH, body, 0)
    plsc.subcore_barrier()
    pltpu.sync_copy(hist, deg_sh.at[idxv], add=True)
    plsc.subcore_barrier()

    @pl.when(sid == 0)
    def _():
        pltpu.sync_copy(deg_sh, deg_out.at[cid])


@functools.partial(
    pl.kernel,
    out_type=jax.ShapeDtypeStruct((NC, NPAD, F), jnp.float32),
    mesh=_mesh,
    scratch_types=[
        pltpu.VMEM((NCH, K), jnp.int32),
        pltpu.VMEM((NCH, K), jnp.int32),
        pltpu.VMEM((K, F), jnp.float32),
        pltpu.VMEM_SHARED((NPAD, F), jnp.float32),
        pltpu.SemaphoreType.DMA,
    ],
)
def _sc_edge_pass(gs_hbm, row_hbm, col_hbm, z_hbm, acc_out,
                  rowv, colv, rbuf, acc_sh, sem):
    cid = lax.axis_index("c")
    sid = lax.axis_index("s")
    wid = sid * NC + cid

    @pl.when(sid == 0)
    def _():
        pltpu.sync_copy(z_hbm, acc_sh)

    pltpu.sync_copy(row_hbm.at[wid], rowv)
    pltpu.sync_copy(col_hbm.at[wid], colv)
    plsc.subcore_barrier()

    def body(j, carry):
        pltpu.async_copy(gs_hbm.at[rowv.at[j]], rbuf, sem).wait()
        pltpu.sync_copy(rbuf, acc_sh.at[colv.at[j]], add=True)
        return carry

    lax.fori_loop(0, NCH, body, 0)
    plsc.subcore_barrier()
    pltpu.sync_copy(
        acc_sh.at[pl.ds(sid * ROWS_PT, ROWS_PT)],
        acc_out.at[cid, pl.ds(sid * ROWS_PT, ROWS_PT)],
    )


BN = 1000


def _tc_scale_matmul(x, d0, d1, Wc, Wl):
    def body(x_ref, d0_ref, d1_ref, wc_ref, wl_ref, gs_ref, dinv_ref):
        deg = d0_ref[...] + d1_ref[...] + 1.0
        dinv = lax.rsqrt(deg)
        g = jnp.dot(x_ref[...], wc_ref[...], preferred_element_type=jnp.float32)
        g = jnp.dot(g, wl_ref[...], preferred_element_type=jnp.float32)
        gs_ref[...] = g * dinv
        dinv_ref[...] = dinv

    return pl.pallas_call(
        body,
        grid=(N // BN,),
        in_specs=[
            pl.BlockSpec((BN, F), lambda i: (i, 0)),
            pl.BlockSpec((BN, 1), lambda i: (i, 0)),
            pl.BlockSpec((BN, 1), lambda i: (i, 0)),
            pl.BlockSpec((F, F), lambda i: (0, 0)),
            pl.BlockSpec((F, F), lambda i: (0, 0)),
        ],
        out_specs=[
            pl.BlockSpec((BN, F), lambda i: (i, 0)),
            pl.BlockSpec((BN, 1), lambda i: (i, 0)),
        ],
        out_shape=[
            jax.ShapeDtypeStruct((N, F), jnp.float32),
            jax.ShapeDtypeStruct((N, 1), jnp.float32),
        ],
    )(x, d0, d1, Wc, Wl)


def _tc_combine(a0, a1, gs, dinv, Wl, bc2, bl2):
    def body(a0_ref, a1_ref, gs_ref, dinv_ref, wl_ref, bc_ref, bl_ref, o_ref):
        acc = a0_ref[...] + a1_ref[...] + gs_ref[...]
        bf = jnp.dot(bc_ref[...], wl_ref[...],
                     preferred_element_type=jnp.float32) + bl_ref[...]
        o_ref[...] = acc * dinv_ref[...] + bf

    return pl.pallas_call(
        body,
        grid=(N // BN,),
        in_specs=[
            pl.BlockSpec((BN, F), lambda i: (i, 0)),
            pl.BlockSpec((BN, F), lambda i: (i, 0)),
            pl.BlockSpec((BN, F), lambda i: (i, 0)),
            pl.BlockSpec((BN, 1), lambda i: (i, 0)),
            pl.BlockSpec((F, F), lambda i: (0, 0)),
            pl.BlockSpec((1, F), lambda i: (0, 0)),
            pl.BlockSpec((1, F), lambda i: (0, 0)),
        ],
        out_specs=pl.BlockSpec((BN, F), lambda i: (i, 0)),
        out_shape=jax.ShapeDtypeStruct((N, F), jnp.float32),
    )(a0, a1, gs, dinv, Wl, bc2, bl2)


@jax.jit
def kernel(x, edge_index, W_conv, b_conv, W_lin, b_lin):
    row = edge_index[0]
    col = edge_index[1]
    pad = PAD_E - E
    rowp = jnp.concatenate(
        [row, jnp.zeros((pad,), jnp.int32)]).reshape(NW, NCH, K)
    colp = jnp.concatenate(
        [col, jnp.full((pad,), N, jnp.int32)]).reshape(NW, NCH, K)

    zdeg = jnp.zeros((DR, 128), jnp.float32)
    zrows = jnp.zeros((NPAD, F), jnp.float32)

    deg2 = _sc_degree(colp, zdeg)
    gs, dinv = _tc_scale_matmul(x, deg2[0].reshape(NPAD, 1),
                                deg2[1].reshape(NPAD, 1), W_conv, W_lin)
    acc2 = _sc_edge_pass(gs, rowp, colp, zrows)
    out = _tc_combine(acc2[0], acc2[1], gs, dinv, W_lin,
                      b_conv.reshape(1, F), b_lin.reshape(1, F))
    return out
